# Initial kernel scaffold; baseline (speedup 1.0000x reference)
import jax
import jax.numpy as jnp
from jax.experimental import pallas as pl


def kernel(x):
    raise NotImplementedError("write your pallas kernel here")


# speedup vs baseline: 10.6459x; 10.6459x over previous
import jax
import jax.numpy as jnp
from jax.experimental import pallas as pl
from jax.experimental.pallas import tpu as pltpu

K_FRAC = 0.1
TEMP = 1.0


def _body(x_ref, o_ref):
    x = x_ref[...]
    rows, d = x.shape
    k = max(1, int(d * K_FRAC))
    u = pltpu.bitcast(x, jnp.uint32)
    neg = (u >> 31) == 1
    u = jnp.where(neg, ~u, u | jnp.uint32(0x80000000))
    prefix = jnp.zeros((rows, 1), jnp.uint32)
    for bit in range(31, -1, -1):
        cand = prefix | jnp.uint32(1 << bit)
        cnt = jnp.sum((u >= cand).astype(jnp.int32), axis=1, keepdims=True)
        prefix = jnp.where(cnt >= k, cand, prefix)
    tneg = (prefix >> 31) == 0
    tbits = jnp.where(tneg, ~prefix, prefix & jnp.uint32(0x7FFFFFFF))
    t = pltpu.bitcast(tbits, jnp.float32)
    z = (x - t) / TEMP
    o_ref[...] = x * (1.0 / (1.0 + jnp.exp(-z)))


def kernel(x):
    b, s, d = x.shape
    xf = x.reshape(b * s, d)
    rows_blk = 256
    out = pl.pallas_call(
        _body,
        grid=((b * s) // rows_blk,),
        in_specs=[pl.BlockSpec((rows_blk, d), lambda i: (i, 0))],
        out_specs=pl.BlockSpec((rows_blk, d), lambda i: (i, 0)),
        out_shape=jax.ShapeDtypeStruct((b * s, d), jnp.float32),
    )(xf)
    return out.reshape(b, s, d)
